# Initial kernel scaffold; baseline (speedup 1.0000x reference)
import jax
import jax.numpy as jnp
from jax.experimental import pallas as pl


def kernel(inputs, gate_w, gate_b):
    raise NotImplementedError("write your pallas kernel here")


# speedup vs baseline: 3.0421x; 3.0421x over previous
import functools

import jax
import jax.numpy as jnp
from jax import lax
from jax.experimental import pallas as pl
from jax.experimental.pallas import tpu as pltpu
from jax.experimental.pallas import tpu_sc as plsc


def _gate_kernel(x_ref, w_ref, b_ref, idx0_ref, idx1_ref, w0_ref, w1_ref,
                 *, n_tokens, n_lanes):
    i = pl.program_id(0)
    x = x_ref[...]
    w = w_ref[...]
    g = lax.dot_general(x, w, (((1,), (1,)), ((), ())),
                        preferred_element_type=jnp.float32)
    g = g + b_ref[...]
    TT, E = g.shape

    lane = lax.broadcasted_iota(jnp.int32, g.shape, 1)
    m1 = jnp.max(g, axis=1, keepdims=True)
    a1 = jnp.min(jnp.where(g == m1, lane, E), axis=1, keepdims=True)
    gm = jnp.where(lane == a1, -jnp.inf, g)
    m2 = jnp.max(gm, axis=1, keepdims=True)
    a2 = jnp.min(jnp.where(gm == m2, lane, E), axis=1, keepdims=True)

    tok = i * TT + lax.broadcasted_iota(jnp.int32, (TT, 1), 0)
    w0 = 1.0 / (1.0 + jnp.exp(m2 - m1))
    idx0_ref[...] = a1 * n_tokens + tok
    idx1_ref[...] = a2 * n_tokens + tok
    w0_ref[...] = jnp.broadcast_to(w0, (TT, n_lanes))
    w1_ref[...] = jnp.broadcast_to(1.0 - w0, (TT, n_lanes))


def _make_sc_combine(T, D, C, NC, NS, L):
    NW = NC * NS
    tpw = T // NW
    mesh = plsc.VectorSubcoreMesh(core_axis_name="c", subcore_axis_name="s")

    @functools.partial(
        pl.kernel,
        out_type=jax.ShapeDtypeStruct((T, D), jnp.float32),
        mesh=mesh,
        scratch_types=[
            pltpu.VMEM((C,), jnp.int32),
            pltpu.VMEM((C,), jnp.int32),
            pltpu.VMEM((C, L), jnp.float32),
            pltpu.VMEM((C, L), jnp.float32),
            pltpu.VMEM((C, D), jnp.float32),
            pltpu.VMEM((C, D), jnp.float32),
            pltpu.SemaphoreType.DMA,
            pltpu.SemaphoreType.DMA,
        ],
    )
    def sc_combine(table, idx0, idx1, w0, w1, out,
                   idxa_v, idxb_v, wa_v, wb_v, rowsa, rowsb, sema, semb):
        wid = lax.axis_index("s") * NC + lax.axis_index("c")
        base0 = wid * tpw

        def chunk(j, carry):
            base = base0 + j * C
            pltpu.sync_copy(idx0.at[pl.ds(base, C)], idxa_v)
            pltpu.sync_copy(idx1.at[pl.ds(base, C)], idxb_v)
            pltpu.sync_copy(w0.at[pl.ds(base, C)], wa_v)
            pltpu.sync_copy(w1.at[pl.ds(base, C)], wb_v)
            ca = pltpu.async_copy(table.at[idxa_v], rowsa, sema)
            cb = pltpu.async_copy(table.at[idxb_v], rowsb, semb)
            ca.wait()
            cb.wait()
            was = [wa_v[c] for c in range(C)]
            wbs = [wb_v[c] for c in range(C)]

            def body(d, carry2):
                s = pl.ds(d * L, L)
                for c in range(C):
                    rowsa[c, s] = was[c] * rowsa[c, s] + wbs[c] * rowsb[c, s]
                return carry2

            lax.fori_loop(0, D // L, body, 0)
            pltpu.sync_copy(rowsa, out.at[pl.ds(base, C)])
            return carry

        lax.fori_loop(0, tpw // C, chunk, 0)

    return sc_combine


def kernel(inputs, gate_w, gate_b):
    E, T, D = inputs.shape
    TT = 256
    grid = T // TT
    L = 16

    idx0, idx1, w0, w1 = pl.pallas_call(
        functools.partial(_gate_kernel, n_tokens=T, n_lanes=L),
        grid=(grid,),
        in_specs=[
            pl.BlockSpec((TT, D), lambda i: (i, 0)),
            pl.BlockSpec((E, D), lambda i: (0, 0)),
            pl.BlockSpec((1, E), lambda i: (0, 0)),
        ],
        out_specs=[
            pl.BlockSpec((TT, 1), lambda i: (i, 0)),
            pl.BlockSpec((TT, 1), lambda i: (i, 0)),
            pl.BlockSpec((TT, L), lambda i: (i, 0)),
            pl.BlockSpec((TT, L), lambda i: (i, 0)),
        ],
        out_shape=[
            jax.ShapeDtypeStruct((T, 1), jnp.int32),
            jax.ShapeDtypeStruct((T, 1), jnp.int32),
            jax.ShapeDtypeStruct((T, L), jnp.float32),
            jax.ShapeDtypeStruct((T, L), jnp.float32),
        ],
    )(inputs[0], gate_w, gate_b.reshape(1, E))

    idx0 = idx0.reshape(T)
    idx1 = idx1.reshape(T)

    info = plsc.get_sparse_core_info()
    NC, NS = info.num_cores, info.num_subcores
    C = 8
    table = inputs.reshape(E * T, D)
    sc_combine = _make_sc_combine(T, D, C, NC, NS, L)
    return sc_combine(table, idx0, idx1, w0, w1)
